# baseline (speedup 1.0000x reference)
import jax
import jax.numpy as jnp
from jax import lax
from jax.experimental import pallas as pl
from jax.experimental.pallas import tpu as pltpu
from jax.experimental.pallas import tpu_sc as plsc

_NUM_CORES = 2
_NUM_SUBCORES = 16
_NW = _NUM_CORES * _NUM_SUBCORES
_IDXROW = 128
_CHUNK = 64
_NSLOT = 4
_SUB = _IDXROW // _CHUNK
_UNROLL = 16


def _gather_body(table_hbm, idx_hbm, out_hbm, idx_v, rows_v, tab_sh, sg, so):
    sid = lax.axis_index("s")
    wid = sid * _NUM_CORES + lax.axis_index("c")
    nrow = idx_v.shape[0]
    nq = nrow * _SUB
    pltpu.sync_copy(idx_hbm.at[wid], idx_v)
    tab_rows = table_hbm.shape[0] // _NUM_SUBCORES
    pltpu.sync_copy(
        table_hbm.at[pl.ds(sid * tab_rows, tab_rows)],
        tab_sh.at[pl.ds(sid * tab_rows, tab_rows)],
    )
    plsc.subcore_barrier()

    def idx_slice(q, db):
        return idx_v.at[q // _SUB, pl.ds((db % _SUB) * _CHUNK, _CHUNK)]

    def _src(pat):
        return table_hbm if pat % 16 == 15 else tab_sh

    def start_gather(q, db, b):
        pltpu.make_async_copy(
            _src(db).at[idx_slice(q, db)], rows_v.at[b], sg.at[b]
        ).start()

    def wait_gather(q, db, b):
        pltpu.make_async_copy(
            _src(db).at[idx_slice(q, db)], rows_v.at[b], sg.at[b]
        ).wait()

    def start_out(q, b):
        pltpu.make_async_copy(
            rows_v.at[b], out_hbm.at[wid, pl.ds(q * _CHUNK, _CHUNK)], so.at[b]
        ).start()

    def wait_out(b):
        pltpu.make_async_copy(
            rows_v.at[b], out_hbm.at[wid, pl.ds(0, _CHUNK)], so.at[b]
        ).wait()

    for q in range(_NSLOT - 1):
        start_gather(q, q, q)

    def body(q0, carry):
        for db in range(_UNROLL):
            q = q0 + db
            b = db % _NSLOT
            nb = (db + _NSLOT - 1) % _NSLOT

            @pl.when(q + _NSLOT - 1 < nq)
            def _():
                @pl.when(q >= 1)
                def _():
                    wait_out(nb)

                start_gather(q + _NSLOT - 1, db + _NSLOT - 1, nb)

            wait_gather(q, db, b)
            start_out(q, b)
        return carry

    lax.fori_loop(0, nq // _UNROLL, lambda i, c: body(i * _UNROLL, c), 0)
    for b in range(_NSLOT):
        wait_out(b)


def kernel(input_text, encodings_weight):
    batch, seq = input_text.shape
    emb = encodings_weight.shape[1]
    n = batch * seq
    per_w = n // _NW
    nrow = per_w // _IDXROW
    assert per_w * _NW == n and nrow * _IDXROW == per_w
    assert (nrow * _SUB) % _NSLOT == 0 and _NSLOT % _SUB == 0

    idx = input_text.reshape(_NW, nrow, _IDXROW).astype(jnp.int32)
    mesh = plsc.VectorSubcoreMesh(core_axis_name="c", subcore_axis_name="s")
    out = pl.kernel(
        _gather_body,
        out_type=jax.ShapeDtypeStruct((_NW, per_w, emb), jnp.float32),
        mesh=mesh,
        scratch_types=[
            pltpu.VMEM((nrow, _IDXROW), jnp.int32),
            pltpu.VMEM((_NSLOT, _CHUNK, emb), jnp.float32),
            pltpu.VMEM_SHARED((encodings_weight.shape[0], emb), jnp.float32),
            pltpu.SemaphoreType.DMA((_NSLOT,)),
            pltpu.SemaphoreType.DMA((_NSLOT,)),
        ],
    )(encodings_weight, idx)
    return out.reshape(batch, seq, emb)


# speedup vs baseline: 1.0685x; 1.0685x over previous
import jax
import jax.numpy as jnp
from jax import lax
from jax.experimental import pallas as pl
from jax.experimental.pallas import tpu as pltpu
from jax.experimental.pallas import tpu_sc as plsc

_NUM_CORES = 2
_NUM_SUBCORES = 16
_NW = _NUM_CORES * _NUM_SUBCORES
_IDXROW = 128
_CHUNK = 64
_NSLOT = 4
_SUB = _IDXROW // _CHUNK


def _gather_body(table_hbm, idx_hbm, out_hbm, idx_v, rows_v, tab_sh, sg, so):
    sid = lax.axis_index("s")
    wid = sid * _NUM_CORES + lax.axis_index("c")
    nrow = idx_v.shape[0]
    nq = nrow * _SUB
    pltpu.sync_copy(idx_hbm.at[wid], idx_v)
    tab_rows = table_hbm.shape[0] // _NUM_SUBCORES
    pltpu.sync_copy(
        table_hbm.at[pl.ds(sid * tab_rows, tab_rows)],
        tab_sh.at[pl.ds(sid * tab_rows, tab_rows)],
    )
    plsc.subcore_barrier()

    def idx_slice(q, db):
        return idx_v.at[q // _SUB, pl.ds((db % _SUB) * _CHUNK, _CHUNK)]

    def start_gather(q, db, b):
        pltpu.make_async_copy(
            tab_sh.at[idx_slice(q, db)], rows_v.at[b], sg.at[b]
        ).start()

    def wait_gather(q, db, b):
        pltpu.make_async_copy(
            tab_sh.at[idx_slice(q, db)], rows_v.at[b], sg.at[b]
        ).wait()

    def start_out(q, b):
        pltpu.make_async_copy(
            rows_v.at[b], out_hbm.at[wid, pl.ds(q * _CHUNK, _CHUNK)], so.at[b]
        ).start()

    def wait_out(b):
        pltpu.make_async_copy(
            rows_v.at[b], out_hbm.at[wid, pl.ds(0, _CHUNK)], so.at[b]
        ).wait()

    for q in range(_NSLOT - 1):
        start_gather(q, q, q)

    for db in range(_NSLOT):
        nb = (db + _NSLOT - 1) % _NSLOT
        if db >= 1:
            wait_out(nb)
        start_gather(db + _NSLOT - 1, db + _NSLOT - 1, nb)
        wait_gather(db, db, db)
        start_out(db, db)

    def body(q0, carry):
        for db in range(_NSLOT):
            q = q0 + db
            nb = (db + _NSLOT - 1) % _NSLOT
            wait_out(nb)
            start_gather(q + _NSLOT - 1, db + _NSLOT - 1, nb)
            wait_gather(q, db, db)
            start_out(q, db)
        return carry

    lax.fori_loop(
        1, nq // _NSLOT - 1, lambda i, c: body(i * _NSLOT, c), 0
    )

    for db in range(_NSLOT):
        q = nq - _NSLOT + db
        nb = (db + _NSLOT - 1) % _NSLOT
        if db == 0:
            wait_out(nb)
            start_gather(q + _NSLOT - 1, db + _NSLOT - 1, nb)
        wait_gather(q, db, db)
        start_out(q, db)

    for b in range(_NSLOT):
        wait_out(b)


def kernel(input_text, encodings_weight):
    batch, seq = input_text.shape
    emb = encodings_weight.shape[1]
    n = batch * seq
    per_w = n // _NW
    nrow = per_w // _IDXROW
    assert per_w * _NW == n and nrow * _IDXROW == per_w
    assert (nrow * _SUB) % _NSLOT == 0 and _NSLOT % _SUB == 0

    idx = input_text.reshape(_NW, nrow, _IDXROW).astype(jnp.int32)
    mesh = plsc.VectorSubcoreMesh(core_axis_name="c", subcore_axis_name="s")
    out = pl.kernel(
        _gather_body,
        out_type=jax.ShapeDtypeStruct((_NW, per_w, emb), jnp.float32),
        mesh=mesh,
        scratch_types=[
            pltpu.VMEM((nrow, _IDXROW), jnp.int32),
            pltpu.VMEM((_NSLOT, _CHUNK, emb), jnp.float32),
            pltpu.VMEM_SHARED((encodings_weight.shape[0], emb), jnp.float32),
            pltpu.SemaphoreType.DMA((_NSLOT,)),
            pltpu.SemaphoreType.DMA((_NSLOT,)),
        ],
    )(encodings_weight, idx)
    return out.reshape(batch, seq, emb)
